# baseline (speedup 1.0000x reference)
import functools

import jax
import jax.numpy as jnp
from jax import lax
from jax.experimental import pallas as pl
from jax.experimental.pallas import tpu as pltpu
from jax.experimental.pallas import tpu_sc as plsc

_BF = jnp.bfloat16
_F32 = jnp.float32


def _conv_body(ph_ref, w_ref, out_ref, *, taps, n):
    acc = None
    for t, (p, off) in enumerate(taps):
        b = ph_ref[0, p, :, off:off + n]
        contrib = jnp.dot(w_ref[t], b, preferred_element_type=_F32)
        acc = contrib if acc is None else acc + contrib
    out_ref[0] = jnp.maximum(acc, 0.0)


def _conv_call(phases, w_taps, taps, n):
    bsz, pnum, ci, lp = phases.shape
    tnum, co, _ = w_taps.shape
    return pl.pallas_call(
        functools.partial(_conv_body, taps=taps, n=n),
        grid=(bsz,),
        in_specs=[
            pl.BlockSpec((1, pnum, ci, lp), lambda b: (b, 0, 0, 0)),
            pl.BlockSpec((tnum, co, ci), lambda b: (0, 0, 0)),
        ],
        out_specs=pl.BlockSpec((1, co, n), lambda b: (b, 0, 0)),
        out_shape=jax.ShapeDtypeStruct((bsz, co, n), _F32),
    )(phases, w_taps)


def _conv0_body(ph_ref, w_ref, out_ref, *, n):
    parts = []
    for ky in range(7):
        for kx in range(7):
            p = (ky % 4) * 4 + (kx % 4)
            off = (ky // 4) * 98 + (kx // 4)
            parts.append(ph_ref[0, p, :, off:off + n])
    b = jnp.concatenate(parts, axis=0)
    out_ref[0] = jnp.maximum(jnp.dot(w_ref[...], b, preferred_element_type=_F32), 0.0)


def _conv0_call(phases, w_mat, n):
    bsz, pnum, ci, lp = phases.shape
    co, _ = w_mat.shape
    return pl.pallas_call(
        functools.partial(_conv0_body, n=n),
        grid=(bsz,),
        in_specs=[
            pl.BlockSpec((1, pnum, ci, lp), lambda b: (b, 0, 0, 0)),
            pl.BlockSpec(w_mat.shape, lambda b: (0, 0)),
        ],
        out_specs=pl.BlockSpec((1, co, n), lambda b: (b, 0, 0)),
        out_shape=jax.ShapeDtypeStruct((bsz, co, n), _F32),
    )(phases, w_mat)


def _phase_split(xp, s):
    bsz, c, hp, wp = xp.shape
    ph = jnp.stack(
        [xp[:, :, pr::s, pc::s] for pr in range(s) for pc in range(s)], axis=1
    )
    return ph.reshape(bsz, s * s, c, (hp // s) * (wp // s))


def _lane_pad(a, lp):
    return jnp.pad(a, [(0, 0)] * (a.ndim - 1) + [(0, lp - a.shape[-1])])


_NC, _NS = 2, 16
_NW = _NC * _NS


def _gather_rows(table, idx, chunk):
    rows, d = table.shape
    bsz = idx.shape[0]
    b_per_w = bsz // _NW
    nchunks = b_per_w // chunk
    idx3 = idx.reshape(_NW, nchunks, chunk)
    mesh = plsc.VectorSubcoreMesh(core_axis_name="c", subcore_axis_name="s")

    @functools.partial(
        pl.kernel,
        mesh=mesh,
        out_type=jax.ShapeDtypeStruct((bsz, d), _F32),
        scratch_types=[
            pltpu.VMEM((chunk,), jnp.int32),
            pltpu.VMEM((chunk, d), _F32),
            pltpu.SemaphoreType.DMA,
        ],
    )
    def k(table_hbm, idx_hbm, out_hbm, idx_v, rows_v, sem):
        cid = lax.axis_index("c")
        sid = lax.axis_index("s")
        wid = sid * _NC + cid
        for c in range(nchunks):
            pltpu.sync_copy(idx_hbm.at[wid, c], idx_v)
            pltpu.async_copy(table_hbm.at[idx_v], rows_v, sem).wait()
            pltpu.sync_copy(
                rows_v, out_hbm.at[pl.ds(wid * b_per_w + c * chunk, chunk)]
            )

    return k(table, idx3)


def kernel(x, W0, W1, W2, W3, W4, fm0, fm1):
    bsz = x.shape[0]

    xp = jnp.pad(x, ((0, 0), (0, 0), (3, 5), (3, 5)))
    ph0 = _phase_split(xp, 4).astype(_BF)
    a0 = W0.transpose(0, 2, 3, 1).reshape(96, 147).astype(_BF)
    h_slab = _conv0_call(ph0, a0, 96 * 98)
    h = h_slab.reshape(bsz, 96, 96, 98)[..., :96]

    hp = jnp.pad(h, ((0, 0), (0, 0), (1, 1), (1, 1)))
    ph1 = _lane_pad(_phase_split(hp, 2), 2408).astype(_BF)
    w1 = W1.transpose(2, 3, 0, 1).astype(_BF)
    taps1 = [((ky % 2) * 2 + (kx % 2), (ky // 2) * 49 + (kx // 2))
             for ky in range(3) for kx in range(3)]
    f0_slab = _conv_call(ph1, w1.reshape(9, 192, 96), taps1, 48 * 49)
    f0 = f0_slab.reshape(bsz, 192, 48, 49)[..., :48]

    f0p = jnp.pad(f0, ((0, 0), (0, 0), (1, 1), (1, 1)))
    in2 = _lane_pad(f0p.reshape(bsz, 1, 192, 2500), 2504).astype(_BF)
    w2 = W2.transpose(2, 3, 0, 1).astype(_BF)
    taps2 = [(0, ky * 50 + kx) for ky in range(3) for kx in range(3)]
    f1_slab = _conv_call(in2, w2.reshape(9, 192, 192), taps2, 48 * 50)
    f1 = jnp.pad(f1_slab, ((0, 0), (0, 0), (51, 49))).reshape(
        bsz, 192, 50, 50)[:, :, 1:49, 1:49]

    f1p = jnp.pad(f1, ((0, 0), (0, 0), (1, 1), (1, 1)))
    ph3 = _lane_pad(_phase_split(f1p, 2), 632).astype(_BF)
    w3 = W3.transpose(2, 3, 0, 1).astype(_BF)
    taps3 = [((ky % 2) * 2 + (kx % 2), (ky // 2) * 25 + (kx // 2))
             for ky in range(3) for kx in range(3)]
    f2_slab = _conv_call(ph3, w3.reshape(9, 384, 192), taps3, 24 * 25)
    f2 = f2_slab.reshape(bsz, 384, 24, 25)[..., :24]

    f2p = jnp.pad(f2, ((0, 0), (0, 0), (1, 1), (1, 1)))
    in4 = _lane_pad(f2p.reshape(bsz, 1, 384, 676), 680).astype(_BF)
    w4 = W4.transpose(2, 3, 0, 1).astype(_BF)
    taps4 = [(0, ky * 26 + kx) for ky in range(3) for kx in range(3)]
    f3_slab = _conv_call(in4, w4.reshape(9, 384, 384), taps4, 24 * 26)
    f3 = jnp.pad(f3_slab, ((0, 0), (0, 0), (27, 25))).reshape(
        bsz, 384, 26, 26)[:, :, 1:25, 1:25]

    cat0 = jnp.concatenate([f0, f1], axis=1).reshape(bsz * 384, 48 * 48)
    cat1 = jnp.pad(
        jnp.concatenate([f2, f3], axis=1).reshape(bsz * 768, 24 * 24),
        ((0, 0), (0, 64)))
    idx0 = (jnp.arange(bsz, dtype=jnp.int32)[:, None] * 384 + fm0[None, :]
            ).reshape(-1)
    idx1 = (jnp.arange(bsz, dtype=jnp.int32)[:, None] * 768 + fm1[None, :]
            ).reshape(-1)
    out0 = _gather_rows(cat0, idx0, 16).reshape(bsz, fm0.shape[0], 48, 48)
    out1 = _gather_rows(cat1, idx1, 64)[:, :576].reshape(
        bsz, fm1.shape[0], 24, 24)
    return (out0, out1)
